# baseline (speedup 1.0000x reference)
import functools
import math

import jax
import jax.numpy as jnp
from jax import lax
from jax.experimental import pallas as pl
from jax.experimental.pallas import tpu as pltpu
from jax.experimental.pallas import tpu_sc as plsc

RATIO = 0.5
_NW = 32


def _sc_gather_rows(table, idx):
    if table.dtype.itemsize == 2:
        packed = lax.bitcast_convert_type(
            table.reshape(table.shape[0], table.shape[1] // 2, 2), jnp.int32)
        out = _sc_gather_rows(packed, idx)
        return lax.bitcast_convert_type(out, table.dtype).reshape(
            idx.shape[0], table.shape[1])
    V, D = table.shape
    B = idx.shape[0]
    b_per_w = B // _NW
    row_bytes = D * table.dtype.itemsize
    chunk = b_per_w
    while chunk * row_bytes > 262144 and chunk % 2 == 0:
        chunk //= 2
    n_chunks = b_per_w // chunk
    mesh = plsc.VectorSubcoreMesh(core_axis_name="c", subcore_axis_name="s")

    @functools.partial(
        pl.kernel, mesh=mesh,
        out_type=jax.ShapeDtypeStruct((B, D), table.dtype),
        scratch_types=[
            pltpu.VMEM((chunk,), jnp.int32),
            pltpu.VMEM((chunk, D), table.dtype),
            pltpu.SemaphoreType.DMA,
        ],
    )
    def k(table_hbm, idx_hbm, out_hbm, idx_v, rows_v, sem):
        wid = lax.axis_index("s") * 2 + lax.axis_index("c")
        base = wid * b_per_w

        def body(c, carry):
            off = base + c * chunk
            pltpu.sync_copy(idx_hbm.at[pl.ds(off, chunk)], idx_v)
            pltpu.async_copy(table_hbm.at[idx_v], rows_v, sem).wait()
            pltpu.sync_copy(rows_v, out_hbm.at[pl.ds(off, chunk)])
            return carry

        lax.fori_loop(0, n_chunks, body, 0)

    return k(table, idx)


def _mm_body(a_ref, b_ref, o_ref):
    o_ref[...] = jnp.dot(a_ref[...], b_ref[...],
                         preferred_element_type=jnp.float32)


def _mm(a_bf, b_bf, bm=512):
    M, K = a_bf.shape
    bm = min(bm, M)
    _, N = b_bf.shape
    return pl.pallas_call(
        _mm_body,
        grid=(M // bm,),
        in_specs=[pl.BlockSpec((bm, K), lambda i: (i, 0)),
                  pl.BlockSpec((K, N), lambda i: (0, 0))],
        out_specs=pl.BlockSpec((bm, N), lambda i: (i, 0)),
        out_shape=jax.ShapeDtypeStruct((M, N), jnp.float32),
    )(a_bf, b_bf)


def _aa_body(l_ref, rt_ref, c_ref, chat_ref, chatt_ref, rs_ref, *, bm, bn):
    i = pl.program_id(0)
    j = pl.program_id(1)
    acc = lax.dot_general(l_ref[...], rt_ref[...],
                          (((1,), (1,)), ((), ())),
                          preferred_element_type=jnp.float32)
    rid = i * bm + lax.broadcasted_iota(jnp.int32, (bm, bn), 0)
    cid = j * bn + lax.broadcasted_iota(jnp.int32, (bm, bn), 1)
    eye = rid == cid
    c = jnp.where(eye, 0.0, acc)
    c_ref[...] = c
    chat = jnp.where(eye, 1.0, c).astype(jnp.bfloat16)
    chat_ref[...] = chat
    chatt_ref[...] = chat.T
    part = jnp.sum(c, axis=1, keepdims=True)

    @pl.when(j == 0)
    def _():
        rs_ref[...] = part

    @pl.when(j != 0)
    def _():
        rs_ref[...] += part


def _mm_aa(l_bf, rt_bf, bm=512, bn=512):
    M, K = l_bf.shape
    N, _ = rt_bf.shape
    bm = min(bm, M)
    bn = min(bn, N)
    return pl.pallas_call(
        functools.partial(_aa_body, bm=bm, bn=bn),
        grid=(M // bm, N // bn),
        in_specs=[pl.BlockSpec((bm, K), lambda i, j: (i, 0)),
                  pl.BlockSpec((bn, K), lambda i, j: (j, 0))],
        out_specs=[pl.BlockSpec((bm, bn), lambda i, j: (i, j)),
                   pl.BlockSpec((bm, bn), lambda i, j: (i, j)),
                   pl.BlockSpec((bn, bm), lambda i, j: (j, i)),
                   pl.BlockSpec((bm, 1), lambda i, j: (i, 0))],
        out_shape=[jax.ShapeDtypeStruct((M, N), jnp.float32),
                   jax.ShapeDtypeStruct((M, N), jnp.bfloat16),
                   jax.ShapeDtypeStruct((N, M), jnp.bfloat16),
                   jax.ShapeDtypeStruct((M, 1), jnp.float32)],
    )(l_bf, rt_bf)


def _prep_body(a_ref, rs_ref, diag_ref, *, bm, n):
    i = pl.program_id(0)
    a = a_ref[...]
    rid = i * bm + lax.broadcasted_iota(jnp.int32, (bm, n), 0)
    cid = lax.broadcasted_iota(jnp.int32, (bm, n), 1)
    eye = rid == cid
    rs_ref[...] = jnp.sum(a, axis=1, keepdims=True)
    diag_ref[...] = jnp.sum(jnp.where(eye, a, 0.0), axis=1, keepdims=True)


def _prep_rowsum_diag(a, bm=512):
    M = a.shape[0]
    bm = min(bm, M)
    return pl.pallas_call(
        functools.partial(_prep_body, bm=bm, n=M),
        grid=(M // bm,),
        in_specs=[pl.BlockSpec((bm, M), lambda i: (i, 0))],
        out_specs=[pl.BlockSpec((bm, 1), lambda i: (i, 0)),
                   pl.BlockSpec((bm, 1), lambda i: (i, 0))],
        out_shape=[jax.ShapeDtypeStruct((M, 1), jnp.float32),
                   jax.ShapeDtypeStruct((M, 1), jnp.float32)],
    )(a)


def _nc1_body(a_ref, dr_ref, dc_ref, norm_ref, ahat_ref, ahatt_ref, *, bm, n):
    i = pl.program_id(0)
    a = a_ref[...]
    rid = i * bm + lax.broadcasted_iota(jnp.int32, (bm, n), 0)
    cid = lax.broadcasted_iota(jnp.int32, (bm, n), 1)
    eye = rid == cid
    extra = jnp.where(eye & (a == 0.0), 2.0, 0.0)
    hat = a + extra
    norm_ref[...] = ((dr_ref[...] * hat) * dc_ref[...]).astype(jnp.bfloat16)
    ahat = jnp.where(eye, 1.0, a).astype(jnp.bfloat16)
    ahat_ref[...] = ahat
    ahatt_ref[...] = ahat.T


def _norm_cast1(a, dinv, bm=512):
    M = a.shape[0]
    bm = min(bm, M)
    dr = dinv.reshape(M, 1)
    dc = dinv.reshape(1, M)
    return pl.pallas_call(
        functools.partial(_nc1_body, bm=bm, n=M),
        grid=(M // bm,),
        in_specs=[pl.BlockSpec((bm, M), lambda i: (i, 0)),
                  pl.BlockSpec((bm, 1), lambda i: (i, 0)),
                  pl.BlockSpec((1, M), lambda i: (0, 0))],
        out_specs=[pl.BlockSpec((bm, M), lambda i: (i, 0)),
                   pl.BlockSpec((bm, M), lambda i: (i, 0)),
                   pl.BlockSpec((M, bm), lambda i: (0, i))],
        out_shape=[jax.ShapeDtypeStruct((M, M), jnp.bfloat16),
                   jax.ShapeDtypeStruct((M, M), jnp.bfloat16),
                   jax.ShapeDtypeStruct((M, M), jnp.bfloat16)],
    )(a, dr, dc)


def _ncs_body(a_ref, dr_ref, dc_ref, norm_ref, *, bm, n):
    i = pl.program_id(0)
    a = a_ref[...]
    rid = i * bm + lax.broadcasted_iota(jnp.int32, (bm, n), 0)
    cid = lax.broadcasted_iota(jnp.int32, (bm, n), 1)
    hat = a + jnp.where(rid == cid, 2.0, 0.0)
    norm_ref[...] = ((dr_ref[...] * hat) * dc_ref[...]).astype(jnp.bfloat16)


def _norm_cast_pooled(a, dinv, bm=512):
    M = a.shape[0]
    bm = min(bm, M)
    dr = dinv.reshape(M, 1)
    dc = dinv.reshape(1, M)
    return pl.pallas_call(
        functools.partial(_ncs_body, bm=bm, n=M),
        grid=(M // bm,),
        in_specs=[pl.BlockSpec((bm, M), lambda i: (i, 0)),
                  pl.BlockSpec((bm, 1), lambda i: (i, 0)),
                  pl.BlockSpec((1, M), lambda i: (0, 0))],
        out_specs=pl.BlockSpec((bm, M), lambda i: (i, 0)),
        out_shape=jax.ShapeDtypeStruct((M, M), jnp.bfloat16),
    )(a, dr, dc)


def _rank_body(si_ref, sall_ref, o_ref, *, bm, n):
    i = pl.program_id(0)
    s_i = si_ref[...]
    s_all = sall_ref[...]
    gt = (s_all > s_i).astype(jnp.int32)
    idx = lax.broadcasted_iota(jnp.int32, (bm, n), 1)
    my = i * bm + lax.broadcasted_iota(jnp.int32, (bm, n), 0)
    eq = ((s_all == s_i) & (idx < my)).astype(jnp.int32)
    o_ref[...] = jnp.sum(gt + eq, axis=1, keepdims=True)


def _ranks(score, bm=512):
    n = score.shape[0]
    bm = min(bm, n)
    return pl.pallas_call(
        functools.partial(_rank_body, bm=bm, n=n),
        grid=(n // bm,),
        in_specs=[pl.BlockSpec((bm, 1), lambda i: (i, 0)),
                  pl.BlockSpec((1, n), lambda i: (0, 0))],
        out_specs=pl.BlockSpec((bm, 1), lambda i: (i, 0)),
        out_shape=jax.ShapeDtypeStruct((n, 1), jnp.int32),
    )(score.reshape(n, 1), score.reshape(1, n))[:, 0]


def _perm_body(rank_ref, o_ref, *, bm, n):
    r0 = pl.program_id(0) * bm
    ranks = rank_ref[...]
    rblk = r0 + lax.broadcasted_iota(jnp.int32, (bm, n), 0)
    nodeid = lax.broadcasted_iota(jnp.int32, (bm, n), 1)
    o_ref[...] = jnp.sum(jnp.where(ranks == rblk, nodeid, 0),
                         axis=1, keepdims=True)


def _perm_from_ranks(rank, k, bm=512):
    n = rank.shape[0]
    bm = min(bm, k)
    return pl.pallas_call(
        functools.partial(_perm_body, bm=bm, n=n),
        grid=(k // bm,),
        in_specs=[pl.BlockSpec((1, n), lambda i: (0, 0))],
        out_specs=pl.BlockSpec((bm, 1), lambda i: (i, 0)),
        out_shape=jax.ShapeDtypeStruct((k, 1), jnp.int32),
    )(rank.reshape(1, n))[:, 0]


def _dinv(deg):
    return jnp.where(deg > 0.0, 1.0 / jnp.sqrt(deg), 0.0)


def _conv(anorm_bf, x, W, b):
    z = _mm(x.astype(jnp.bfloat16), W.astype(jnp.bfloat16))
    return _mm(anorm_bf, z.astype(jnp.bfloat16)) + b


def _score(x, p):
    n, f = x.shape
    p_pad = jnp.zeros((f, 128), jnp.float32).at[:, 0].set(p)
    s = _mm(x.astype(jnp.bfloat16), p_pad.astype(jnp.bfloat16))[:, 0]
    return s / jnp.linalg.norm(p)


def _pool(xc, s, ahat_bf, ahatt_bf):
    n = s.shape[0]
    k = n // 2
    rank = _ranks(s)
    perm = _perm_from_ranks(rank, k)
    xn = (xc * jnp.tanh(s)[:, None])[perm]
    C, Chat, ChatT, rs = _mm_aa(_sc_gather_rows(ahat_bf, perm),
                                _sc_gather_rows(ahatt_bf, perm))
    deg = rs[:, 0] + 2.0
    return xn, rank, perm, C, Chat, ChatT, deg


def kernel(x, edge_index, W_d1, b_d1, W_d2, b_d2, W_u1, b_u1, W_u2, b_u2,
           W_u3, b_u3, p1, p2, p3):
    N = x.shape[0]

    A1 = jnp.zeros((N, N), jnp.float32).at[edge_index[1], edge_index[0]].add(1.0)
    rs1, diag1 = _prep_rowsum_diag(A1)
    deg1 = rs1[:, 0] + jnp.where(diag1[:, 0] == 0.0, 2.0, 0.0)
    Anorm1, Ahat1, Ahat1T = _norm_cast1(A1, _dinv(deg1))

    x1 = jax.nn.elu(_conv(Anorm1, x, W_d1, b_d1))

    x2, rank1, perm1, A2, Ahat2, Ahat2T, deg2 = _pool(x1, _score(x1, p1),
                                                      Ahat1, Ahat1T)
    Anorm2 = _norm_cast_pooled(A2, _dinv(deg2))
    x2 = jax.nn.elu(_conv(Anorm2, x2, W_d1, b_d1))

    x3, rank2, perm2, A3, Ahat3, Ahat3T, deg3 = _pool(x2, _score(x2, p2),
                                                      Ahat2, Ahat2T)
    Anorm3 = _norm_cast_pooled(A3, _dinv(deg3))
    x3 = jax.nn.elu(_conv(Anorm3, x3, W_d2, b_d2))

    x4, rank3, perm3, A4, _, _, deg4 = _pool(x3, _score(x3, p3),
                                             Ahat3, Ahat3T)
    Anorm4 = _norm_cast_pooled(A4, _dinv(deg4))
    x4 = jax.nn.elu(_conv(Anorm4, x4, W_d2, b_d2))

    def unpool(xk, rank, k):
        idx = jnp.minimum(rank, k - 1)
        return jnp.where((rank < k)[:, None], xk[idx], 0.0)

    x3 = x3 + unpool(x4, rank3, N // 8)
    x3 = jax.nn.elu(_conv(Anorm3, x3, W_u1, b_u1))
    x2 = x2 + unpool(x3, rank2, N // 4)
    x2 = jax.nn.elu(_conv(Anorm2, x2, W_u2, b_u2))
    x1 = x1 + unpool(x2, rank1, N // 2)
    out = _conv(Anorm1, x1, W_u3, b_u3)
    return out


# speedup vs baseline: 1.7947x; 1.7947x over previous
import functools
import math

import jax
import jax.numpy as jnp
from jax import lax
from jax.experimental import pallas as pl
from jax.experimental.pallas import tpu as pltpu
from jax.experimental.pallas import tpu_sc as plsc

RATIO = 0.5
_NW = 32


def _sc_gather_rows(table, idx):
    V, D = table.shape
    B = idx.shape[0]
    b_per_w = B // _NW
    chunk = b_per_w
    while chunk * D * 4 > 262144 and chunk % 2 == 0:
        chunk //= 2
    n_chunks = b_per_w // chunk
    mesh = plsc.VectorSubcoreMesh(core_axis_name="c", subcore_axis_name="s")

    @functools.partial(
        pl.kernel, mesh=mesh,
        out_type=jax.ShapeDtypeStruct((B, D), table.dtype),
        scratch_types=[
            pltpu.VMEM((chunk,), jnp.int32),
            pltpu.VMEM((chunk, D), table.dtype),
            pltpu.SemaphoreType.DMA,
        ],
    )
    def k(table_hbm, idx_hbm, out_hbm, idx_v, rows_v, sem):
        wid = lax.axis_index("s") * 2 + lax.axis_index("c")
        base = wid * b_per_w

        def body(c, carry):
            off = base + c * chunk
            pltpu.sync_copy(idx_hbm.at[pl.ds(off, chunk)], idx_v)
            pltpu.async_copy(table_hbm.at[idx_v], rows_v, sem).wait()
            pltpu.sync_copy(rows_v, out_hbm.at[pl.ds(off, chunk)])
            return carry

        lax.fori_loop(0, n_chunks, body, 0)

    return k(table, idx)


def _mm_body(a_ref, b_ref, o_ref):
    o_ref[...] = jnp.dot(a_ref[...], b_ref[...],
                         preferred_element_type=jnp.float32)


def _mm(a_bf, b_bf, bm=512):
    M, K = a_bf.shape
    bm = min(bm, M)
    _, N = b_bf.shape
    return pl.pallas_call(
        _mm_body,
        grid=(M // bm,),
        in_specs=[pl.BlockSpec((bm, K), lambda i: (i, 0)),
                  pl.BlockSpec((K, N), lambda i: (0, 0))],
        out_specs=pl.BlockSpec((bm, N), lambda i: (i, 0)),
        out_shape=jax.ShapeDtypeStruct((M, N), jnp.float32),
    )(a_bf, b_bf)


def _aa_body(l_ref, rt_ref, c_ref, chat_ref, chatt_ref, rs_ref, *, bm, bn):
    i = pl.program_id(0)
    j = pl.program_id(1)
    acc = lax.dot_general(l_ref[...], rt_ref[...],
                          (((1,), (1,)), ((), ())),
                          preferred_element_type=jnp.float32)
    rid = i * bm + lax.broadcasted_iota(jnp.int32, (bm, bn), 0)
    cid = j * bn + lax.broadcasted_iota(jnp.int32, (bm, bn), 1)
    eye = rid == cid
    c = jnp.where(eye, 0.0, acc)
    c_ref[...] = c
    chat = jnp.where(eye, 1.0, c).astype(jnp.bfloat16)
    chat_ref[...] = chat
    chatt_ref[...] = chat.T
    part = jnp.sum(c, axis=1, keepdims=True)

    @pl.when(j == 0)
    def _():
        rs_ref[...] = part

    @pl.when(j != 0)
    def _():
        rs_ref[...] += part


def _mm_aa(l_bf, rt_bf, bm=1024, bn=512):
    M, K = l_bf.shape
    N, _ = rt_bf.shape
    bm = min(bm, M)
    bn = min(bn, N)
    return pl.pallas_call(
        functools.partial(_aa_body, bm=bm, bn=bn),
        grid=(M // bm, N // bn),
        in_specs=[pl.BlockSpec((bm, K), lambda i, j: (i, 0)),
                  pl.BlockSpec((bn, K), lambda i, j: (j, 0))],
        out_specs=[pl.BlockSpec((bm, bn), lambda i, j: (i, j)),
                   pl.BlockSpec((bm, bn), lambda i, j: (i, j)),
                   pl.BlockSpec((bn, bm), lambda i, j: (j, i)),
                   pl.BlockSpec((bm, 1), lambda i, j: (i, 0))],
        out_shape=[jax.ShapeDtypeStruct((M, N), jnp.float32),
                   jax.ShapeDtypeStruct((M, N), jnp.bfloat16),
                   jax.ShapeDtypeStruct((N, M), jnp.bfloat16),
                   jax.ShapeDtypeStruct((M, 1), jnp.float32)],
    )(l_bf, rt_bf)


def _prep_body(a_ref, rs_ref, diag_ref, *, bm, n):
    i = pl.program_id(0)
    a = a_ref[...]
    rid = i * bm + lax.broadcasted_iota(jnp.int32, (bm, n), 0)
    cid = lax.broadcasted_iota(jnp.int32, (bm, n), 1)
    eye = rid == cid
    rs_ref[...] = jnp.sum(a, axis=1, keepdims=True)
    diag_ref[...] = jnp.sum(jnp.where(eye, a, 0.0), axis=1, keepdims=True)


def _prep_rowsum_diag(a, bm=512):
    M = a.shape[0]
    bm = min(bm, M)
    return pl.pallas_call(
        functools.partial(_prep_body, bm=bm, n=M),
        grid=(M // bm,),
        in_specs=[pl.BlockSpec((bm, M), lambda i: (i, 0))],
        out_specs=[pl.BlockSpec((bm, 1), lambda i: (i, 0)),
                   pl.BlockSpec((bm, 1), lambda i: (i, 0))],
        out_shape=[jax.ShapeDtypeStruct((M, 1), jnp.float32),
                   jax.ShapeDtypeStruct((M, 1), jnp.float32)],
    )(a)


def _nc1_body(a_ref, dr_ref, dc_ref, norm_ref, ahat_ref, ahatt_ref, *, bm, n):
    i = pl.program_id(0)
    a = a_ref[...]
    rid = i * bm + lax.broadcasted_iota(jnp.int32, (bm, n), 0)
    cid = lax.broadcasted_iota(jnp.int32, (bm, n), 1)
    eye = rid == cid
    extra = jnp.where(eye & (a == 0.0), 2.0, 0.0)
    hat = a + extra
    norm_ref[...] = ((dr_ref[...] * hat) * dc_ref[...]).astype(jnp.bfloat16)
    ahat = jnp.where(eye, 1.0, a).astype(jnp.bfloat16)
    ahat_ref[...] = ahat
    ahatt_ref[...] = ahat.T


def _norm_cast1(a, dinv, bm=512):
    M = a.shape[0]
    bm = min(bm, M)
    dr = dinv.reshape(M, 1)
    dc = dinv.reshape(1, M)
    return pl.pallas_call(
        functools.partial(_nc1_body, bm=bm, n=M),
        grid=(M // bm,),
        in_specs=[pl.BlockSpec((bm, M), lambda i: (i, 0)),
                  pl.BlockSpec((bm, 1), lambda i: (i, 0)),
                  pl.BlockSpec((1, M), lambda i: (0, 0))],
        out_specs=[pl.BlockSpec((bm, M), lambda i: (i, 0)),
                   pl.BlockSpec((bm, M), lambda i: (i, 0)),
                   pl.BlockSpec((M, bm), lambda i: (0, i))],
        out_shape=[jax.ShapeDtypeStruct((M, M), jnp.bfloat16),
                   jax.ShapeDtypeStruct((M, M), jnp.bfloat16),
                   jax.ShapeDtypeStruct((M, M), jnp.bfloat16)],
    )(a, dr, dc)


def _ncs_body(a_ref, dr_ref, dc_ref, norm_ref, *, bm, n):
    i = pl.program_id(0)
    a = a_ref[...]
    rid = i * bm + lax.broadcasted_iota(jnp.int32, (bm, n), 0)
    cid = lax.broadcasted_iota(jnp.int32, (bm, n), 1)
    hat = a + jnp.where(rid == cid, 2.0, 0.0)
    norm_ref[...] = ((dr_ref[...] * hat) * dc_ref[...]).astype(jnp.bfloat16)


def _norm_cast_pooled(a, dinv, bm=512):
    M = a.shape[0]
    bm = min(bm, M)
    dr = dinv.reshape(M, 1)
    dc = dinv.reshape(1, M)
    return pl.pallas_call(
        functools.partial(_ncs_body, bm=bm, n=M),
        grid=(M // bm,),
        in_specs=[pl.BlockSpec((bm, M), lambda i: (i, 0)),
                  pl.BlockSpec((bm, 1), lambda i: (i, 0)),
                  pl.BlockSpec((1, M), lambda i: (0, 0))],
        out_specs=pl.BlockSpec((bm, M), lambda i: (i, 0)),
        out_shape=jax.ShapeDtypeStruct((M, M), jnp.bfloat16),
    )(a, dr, dc)


def _rank_body(si_ref, sall_ref, o_ref, *, bm, n):
    i = pl.program_id(0)
    s_i = si_ref[...]
    s_all = sall_ref[...]
    gt = (s_all > s_i).astype(jnp.int32)
    idx = lax.broadcasted_iota(jnp.int32, (bm, n), 1)
    my = i * bm + lax.broadcasted_iota(jnp.int32, (bm, n), 0)
    eq = ((s_all == s_i) & (idx < my)).astype(jnp.int32)
    o_ref[...] = jnp.sum(gt + eq, axis=1, keepdims=True)


def _ranks(score, bm=512):
    n = score.shape[0]
    bm = min(bm, n)
    return pl.pallas_call(
        functools.partial(_rank_body, bm=bm, n=n),
        grid=(n // bm,),
        in_specs=[pl.BlockSpec((bm, 1), lambda i: (i, 0)),
                  pl.BlockSpec((1, n), lambda i: (0, 0))],
        out_specs=pl.BlockSpec((bm, 1), lambda i: (i, 0)),
        out_shape=jax.ShapeDtypeStruct((n, 1), jnp.int32),
    )(score.reshape(n, 1), score.reshape(1, n))[:, 0]


def _perm_body(rank_ref, o_ref, *, bm, n):
    r0 = pl.program_id(0) * bm
    ranks = rank_ref[...]
    rblk = r0 + lax.broadcasted_iota(jnp.int32, (bm, n), 0)
    nodeid = lax.broadcasted_iota(jnp.int32, (bm, n), 1)
    o_ref[...] = jnp.sum(jnp.where(ranks == rblk, nodeid, 0),
                         axis=1, keepdims=True)


def _perm_from_ranks(rank, k, bm=512):
    n = rank.shape[0]
    bm = min(bm, k)
    return pl.pallas_call(
        functools.partial(_perm_body, bm=bm, n=n),
        grid=(k // bm,),
        in_specs=[pl.BlockSpec((1, n), lambda i: (0, 0))],
        out_specs=pl.BlockSpec((bm, 1), lambda i: (i, 0)),
        out_shape=jax.ShapeDtypeStruct((k, 1), jnp.int32),
    )(rank.reshape(1, n))[:, 0]


def _dinv(deg):
    return jnp.where(deg > 0.0, 1.0 / jnp.sqrt(deg), 0.0)


def _conv(anorm_bf, x, W, b):
    z = _mm(x.astype(jnp.bfloat16), W.astype(jnp.bfloat16))
    return _mm(anorm_bf, z.astype(jnp.bfloat16)) + b


def _score(x, p):
    n, f = x.shape
    p_pad = jnp.zeros((f, 128), jnp.float32).at[:, 0].set(p)
    s = _mm(x.astype(jnp.bfloat16), p_pad.astype(jnp.bfloat16))[:, 0]
    return s / jnp.linalg.norm(p)


def _pool(xc, s, ahat_bf, ahatt_bf):
    n = s.shape[0]
    k = n // 2
    rank = _ranks(s)
    perm = _perm_from_ranks(rank, k)
    xn = _sc_gather_rows(xc * jnp.tanh(s)[:, None], perm)
    C, Chat, ChatT, rs = _mm_aa(ahat_bf[perm], ahatt_bf[perm])
    deg = rs[:, 0] + 2.0
    return xn, rank, perm, C, Chat, ChatT, deg


def kernel(x, edge_index, W_d1, b_d1, W_d2, b_d2, W_u1, b_u1, W_u2, b_u2,
           W_u3, b_u3, p1, p2, p3):
    N = x.shape[0]

    A1 = jnp.zeros((N, N), jnp.float32).at[edge_index[1], edge_index[0]].add(1.0)
    rs1, diag1 = _prep_rowsum_diag(A1)
    deg1 = rs1[:, 0] + jnp.where(diag1[:, 0] == 0.0, 2.0, 0.0)
    Anorm1, Ahat1, Ahat1T = _norm_cast1(A1, _dinv(deg1))

    x1 = jax.nn.elu(_conv(Anorm1, x, W_d1, b_d1))

    x2, rank1, perm1, A2, Ahat2, Ahat2T, deg2 = _pool(x1, _score(x1, p1),
                                                      Ahat1, Ahat1T)
    Anorm2 = _norm_cast_pooled(A2, _dinv(deg2))
    x2 = jax.nn.elu(_conv(Anorm2, x2, W_d1, b_d1))

    x3, rank2, perm2, A3, Ahat3, Ahat3T, deg3 = _pool(x2, _score(x2, p2),
                                                      Ahat2, Ahat2T)
    Anorm3 = _norm_cast_pooled(A3, _dinv(deg3))
    x3 = jax.nn.elu(_conv(Anorm3, x3, W_d2, b_d2))

    x4, rank3, perm3, A4, _, _, deg4 = _pool(x3, _score(x3, p3),
                                             Ahat3, Ahat3T)
    Anorm4 = _norm_cast_pooled(A4, _dinv(deg4))
    x4 = jax.nn.elu(_conv(Anorm4, x4, W_d2, b_d2))

    def unpool(xk, rank, k):
        idx = jnp.minimum(rank, k - 1)
        return jnp.where((rank < k)[:, None], _sc_gather_rows(xk, idx), 0.0)

    x3 = x3 + unpool(x4, rank3, N // 8)
    x3 = jax.nn.elu(_conv(Anorm3, x3, W_u1, b_u1))
    x2 = x2 + unpool(x3, rank2, N // 4)
    x2 = jax.nn.elu(_conv(Anorm2, x2, W_u2, b_u2))
    x1 = x1 + unpool(x2, rank1, N // 2)
    out = _conv(Anorm1, x1, W_u3, b_u3)
    return out


# speedup vs baseline: 2.0394x; 1.1363x over previous
import functools
import math

import jax
import jax.numpy as jnp
from jax import lax
from jax.experimental import pallas as pl
from jax.experimental.pallas import tpu as pltpu
from jax.experimental.pallas import tpu_sc as plsc

RATIO = 0.5
_NW = 32


def _sc_gather_rows(table, idx):
    V, D = table.shape
    B = idx.shape[0]
    b_per_w = B // _NW
    chunk = b_per_w
    while chunk * D * 4 > 262144 and chunk % 2 == 0:
        chunk //= 2
    n_chunks = b_per_w // chunk
    mesh = plsc.VectorSubcoreMesh(core_axis_name="c", subcore_axis_name="s")

    @functools.partial(
        pl.kernel, mesh=mesh,
        out_type=jax.ShapeDtypeStruct((B, D), table.dtype),
        scratch_types=[
            pltpu.VMEM((chunk,), jnp.int32),
            pltpu.VMEM((chunk, D), table.dtype),
            pltpu.SemaphoreType.DMA,
        ],
    )
    def k(table_hbm, idx_hbm, out_hbm, idx_v, rows_v, sem):
        wid = lax.axis_index("s") * 2 + lax.axis_index("c")
        base = wid * b_per_w

        def body(c, carry):
            off = base + c * chunk
            pltpu.sync_copy(idx_hbm.at[pl.ds(off, chunk)], idx_v)
            pltpu.async_copy(table_hbm.at[idx_v], rows_v, sem).wait()
            pltpu.sync_copy(rows_v, out_hbm.at[pl.ds(off, chunk)])
            return carry

        lax.fori_loop(0, n_chunks, body, 0)

    return k(table, idx)


def _mm_body(a_ref, b_ref, o_ref):
    o_ref[...] = jnp.dot(a_ref[...], b_ref[...],
                         preferred_element_type=jnp.float32)


def _mm(a_bf, b_bf, bm=512):
    M, K = a_bf.shape
    bm = min(bm, M)
    _, N = b_bf.shape
    return pl.pallas_call(
        _mm_body,
        grid=(M // bm,),
        in_specs=[pl.BlockSpec((bm, K), lambda i: (i, 0)),
                  pl.BlockSpec((K, N), lambda i: (0, 0))],
        out_specs=pl.BlockSpec((bm, N), lambda i: (i, 0)),
        out_shape=jax.ShapeDtypeStruct((M, N), jnp.float32),
    )(a_bf, b_bf)


def _aa_body(l_ref, rt_ref, c_ref, chat_ref, chatt_ref, rs_ref, *, bm, bn):
    i = pl.program_id(0)
    j = pl.program_id(1)
    acc = lax.dot_general(l_ref[...], rt_ref[...],
                          (((1,), (1,)), ((), ())),
                          preferred_element_type=jnp.float32)
    rid = i * bm + lax.broadcasted_iota(jnp.int32, (bm, bn), 0)
    cid = j * bn + lax.broadcasted_iota(jnp.int32, (bm, bn), 1)
    eye = rid == cid
    c = jnp.where(eye, 0.0, acc)
    c_ref[...] = c
    chat = jnp.where(eye, 1.0, c).astype(jnp.bfloat16)
    chat_ref[...] = chat
    chatt_ref[...] = chat.T
    part = jnp.sum(c, axis=1, keepdims=True)

    @pl.when(j == 0)
    def _():
        rs_ref[...] = part

    @pl.when(j != 0)
    def _():
        rs_ref[...] += part


def _mm_aa(l_bf, rt_bf, bm=1024, bn=512):
    M, K = l_bf.shape
    N, _ = rt_bf.shape
    bm = min(bm, M)
    bn = min(bn, N)
    return pl.pallas_call(
        functools.partial(_aa_body, bm=bm, bn=bn),
        grid=(M // bm, N // bn),
        in_specs=[pl.BlockSpec((bm, K), lambda i, j: (i, 0)),
                  pl.BlockSpec((bn, K), lambda i, j: (j, 0))],
        out_specs=[pl.BlockSpec((bm, bn), lambda i, j: (i, j)),
                   pl.BlockSpec((bm, bn), lambda i, j: (i, j)),
                   pl.BlockSpec((bn, bm), lambda i, j: (j, i)),
                   pl.BlockSpec((bm, 1), lambda i, j: (i, 0))],
        out_shape=[jax.ShapeDtypeStruct((M, N), jnp.float32),
                   jax.ShapeDtypeStruct((M, N), jnp.bfloat16),
                   jax.ShapeDtypeStruct((N, M), jnp.bfloat16),
                   jax.ShapeDtypeStruct((M, 1), jnp.float32)],
    )(l_bf, rt_bf)


def _prep_body(a_ref, rs_ref, diag_ref, *, bm, n):
    i = pl.program_id(0)
    a = a_ref[...]
    rid = i * bm + lax.broadcasted_iota(jnp.int32, (bm, n), 0)
    cid = lax.broadcasted_iota(jnp.int32, (bm, n), 1)
    eye = rid == cid
    rs_ref[...] = jnp.sum(a, axis=1, keepdims=True)
    diag_ref[...] = jnp.sum(jnp.where(eye, a, 0.0), axis=1, keepdims=True)


def _prep_rowsum_diag(a, bm=512):
    M = a.shape[0]
    bm = min(bm, M)
    return pl.pallas_call(
        functools.partial(_prep_body, bm=bm, n=M),
        grid=(M // bm,),
        in_specs=[pl.BlockSpec((bm, M), lambda i: (i, 0))],
        out_specs=[pl.BlockSpec((bm, 1), lambda i: (i, 0)),
                   pl.BlockSpec((bm, 1), lambda i: (i, 0))],
        out_shape=[jax.ShapeDtypeStruct((M, 1), jnp.float32),
                   jax.ShapeDtypeStruct((M, 1), jnp.float32)],
    )(a)


def _nc1_body(a_ref, dr_ref, dc_ref, norm_ref, ahat_ref, ahatt_ref, *, bm, n):
    i = pl.program_id(0)
    a = a_ref[...]
    rid = i * bm + lax.broadcasted_iota(jnp.int32, (bm, n), 0)
    cid = lax.broadcasted_iota(jnp.int32, (bm, n), 1)
    eye = rid == cid
    extra = jnp.where(eye & (a == 0.0), 2.0, 0.0)
    hat = a + extra
    norm_ref[...] = ((dr_ref[...] * hat) * dc_ref[...]).astype(jnp.bfloat16)
    ahat = jnp.where(eye, 1.0, a).astype(jnp.bfloat16)
    ahat_ref[...] = ahat
    ahatt_ref[...] = ahat.T


def _norm_cast1(a, dinv, bm=512):
    M = a.shape[0]
    bm = min(bm, M)
    dr = dinv.reshape(M, 1)
    dc = dinv.reshape(1, M)
    return pl.pallas_call(
        functools.partial(_nc1_body, bm=bm, n=M),
        grid=(M // bm,),
        in_specs=[pl.BlockSpec((bm, M), lambda i: (i, 0)),
                  pl.BlockSpec((bm, 1), lambda i: (i, 0)),
                  pl.BlockSpec((1, M), lambda i: (0, 0))],
        out_specs=[pl.BlockSpec((bm, M), lambda i: (i, 0)),
                   pl.BlockSpec((bm, M), lambda i: (i, 0)),
                   pl.BlockSpec((M, bm), lambda i: (0, i))],
        out_shape=[jax.ShapeDtypeStruct((M, M), jnp.bfloat16),
                   jax.ShapeDtypeStruct((M, M), jnp.bfloat16),
                   jax.ShapeDtypeStruct((M, M), jnp.bfloat16)],
    )(a, dr, dc)


def _ncs_body(a_ref, dr_ref, dc_ref, norm_ref, *, bm, n):
    i = pl.program_id(0)
    a = a_ref[...]
    rid = i * bm + lax.broadcasted_iota(jnp.int32, (bm, n), 0)
    cid = lax.broadcasted_iota(jnp.int32, (bm, n), 1)
    hat = a + jnp.where(rid == cid, 2.0, 0.0)
    norm_ref[...] = ((dr_ref[...] * hat) * dc_ref[...]).astype(jnp.bfloat16)


def _norm_cast_pooled(a, dinv, bm=512):
    M = a.shape[0]
    bm = min(bm, M)
    dr = dinv.reshape(M, 1)
    dc = dinv.reshape(1, M)
    return pl.pallas_call(
        functools.partial(_ncs_body, bm=bm, n=M),
        grid=(M // bm,),
        in_specs=[pl.BlockSpec((bm, M), lambda i: (i, 0)),
                  pl.BlockSpec((bm, 1), lambda i: (i, 0)),
                  pl.BlockSpec((1, M), lambda i: (0, 0))],
        out_specs=pl.BlockSpec((bm, M), lambda i: (i, 0)),
        out_shape=jax.ShapeDtypeStruct((M, M), jnp.bfloat16),
    )(a, dr, dc)


def _rank_body(si_ref, sall_ref, o_ref, *, bm, n):
    i = pl.program_id(0)
    s_i = si_ref[...]
    s_all = sall_ref[...]
    gt = (s_all > s_i).astype(jnp.int32)
    idx = lax.broadcasted_iota(jnp.int32, (bm, n), 1)
    my = i * bm + lax.broadcasted_iota(jnp.int32, (bm, n), 0)
    eq = ((s_all == s_i) & (idx < my)).astype(jnp.int32)
    o_ref[...] = jnp.sum(gt + eq, axis=1, keepdims=True)


def _ranks(score, bm=512):
    n = score.shape[0]
    bm = min(bm, n)
    return pl.pallas_call(
        functools.partial(_rank_body, bm=bm, n=n),
        grid=(n // bm,),
        in_specs=[pl.BlockSpec((bm, 1), lambda i: (i, 0)),
                  pl.BlockSpec((1, n), lambda i: (0, 0))],
        out_specs=pl.BlockSpec((bm, 1), lambda i: (i, 0)),
        out_shape=jax.ShapeDtypeStruct((n, 1), jnp.int32),
    )(score.reshape(n, 1), score.reshape(1, n))[:, 0]


def _perm_body(rank_ref, o_ref, *, bm, n):
    r0 = pl.program_id(0) * bm
    ranks = rank_ref[...]
    rblk = r0 + lax.broadcasted_iota(jnp.int32, (bm, n), 0)
    nodeid = lax.broadcasted_iota(jnp.int32, (bm, n), 1)
    o_ref[...] = jnp.sum(jnp.where(ranks == rblk, nodeid, 0),
                         axis=1, keepdims=True)


def _perm_from_ranks(rank, k, bm=512):
    n = rank.shape[0]
    bm = min(bm, k)
    return pl.pallas_call(
        functools.partial(_perm_body, bm=bm, n=n),
        grid=(k // bm,),
        in_specs=[pl.BlockSpec((1, n), lambda i: (0, 0))],
        out_specs=pl.BlockSpec((bm, 1), lambda i: (i, 0)),
        out_shape=jax.ShapeDtypeStruct((k, 1), jnp.int32),
    )(rank.reshape(1, n))[:, 0]


def _dinv(deg):
    return jnp.where(deg > 0.0, 1.0 / jnp.sqrt(deg), 0.0)


def _conv(anorm_bf, x, W, b):
    z = _mm(x.astype(jnp.bfloat16), W.astype(jnp.bfloat16))
    return _mm(anorm_bf, z.astype(jnp.bfloat16)) + b


def _score(x, p):
    n, f = x.shape
    p_pad = jnp.zeros((f, 128), jnp.float32).at[:, 0].set(p)
    s = _mm(x.astype(jnp.bfloat16), p_pad.astype(jnp.bfloat16))[:, 0]
    return s / jnp.linalg.norm(p)


def _pool(xc, s, ahat_bf, ahatt_bf):
    n = s.shape[0]
    k = n // 2
    rank = _ranks(s)
    perm = _perm_from_ranks(rank, k)
    xn = (xc * jnp.tanh(s)[:, None])[perm]
    C, Chat, ChatT, rs = _mm_aa(ahat_bf[perm], ahatt_bf[perm])
    deg = rs[:, 0] + 2.0
    return xn, rank, perm, C, Chat, ChatT, deg


def kernel(x, edge_index, W_d1, b_d1, W_d2, b_d2, W_u1, b_u1, W_u2, b_u2,
           W_u3, b_u3, p1, p2, p3):
    N = x.shape[0]

    A1 = jnp.zeros((N, N), jnp.float32).at[edge_index[1], edge_index[0]].add(1.0)
    rs1, diag1 = _prep_rowsum_diag(A1)
    deg1 = rs1[:, 0] + jnp.where(diag1[:, 0] == 0.0, 2.0, 0.0)
    Anorm1, Ahat1, Ahat1T = _norm_cast1(A1, _dinv(deg1))

    x1 = jax.nn.elu(_conv(Anorm1, x, W_d1, b_d1))

    x2, rank1, perm1, A2, Ahat2, Ahat2T, deg2 = _pool(x1, _score(x1, p1),
                                                      Ahat1, Ahat1T)
    Anorm2 = _norm_cast_pooled(A2, _dinv(deg2))
    x2 = jax.nn.elu(_conv(Anorm2, x2, W_d1, b_d1))

    x3, rank2, perm2, A3, Ahat3, Ahat3T, deg3 = _pool(x2, _score(x2, p2),
                                                      Ahat2, Ahat2T)
    Anorm3 = _norm_cast_pooled(A3, _dinv(deg3))
    x3 = jax.nn.elu(_conv(Anorm3, x3, W_d2, b_d2))

    x4, rank3, perm3, A4, _, _, deg4 = _pool(x3, _score(x3, p3),
                                             Ahat3, Ahat3T)
    Anorm4 = _norm_cast_pooled(A4, _dinv(deg4))
    x4 = jax.nn.elu(_conv(Anorm4, x4, W_d2, b_d2))

    def unpool(xk, rank, k):
        idx = jnp.minimum(rank, k - 1)
        return jnp.where((rank < k)[:, None], xk[idx], 0.0)

    x3 = x3 + unpool(x4, rank3, N // 8)
    x3 = jax.nn.elu(_conv(Anorm3, x3, W_u1, b_u1))
    x2 = x2 + unpool(x3, rank2, N // 4)
    x2 = jax.nn.elu(_conv(Anorm2, x2, W_u2, b_u2))
    x1 = x1 + unpool(x2, rank1, N // 2)
    out = _conv(Anorm1, x1, W_u3, b_u3)
    return out
